# baseline (speedup 1.0000x reference)
import functools

import jax
import jax.numpy as jnp
from jax.experimental import pallas as pl
from jax.experimental.pallas import tpu as pltpu
from jax.experimental.pallas import tpu_sc as plsc

D = 128
H = 6
NA = 384
NL = 640
B = 8
NA_PER = 48
NL_PER = 80
N = NA + NL
NPS = NA_PER + NL_PER
E = B * NPS

PH = jax.lax.Precision.DEFAULT

_SC_INFO = plsc.get_sparse_core_info()
NC = _SC_INFO.num_cores
NS = _SC_INFO.num_subcores
NW = NC * NS
RPW = E // NW
NA_W = (B * NA_PER) // RPW
_SC_MESH = plsc.VectorSubcoreMesh(core_axis_name="c", subcore_axis_name="s")


def _sc_out_gather_body(y_hbm, aidx_hbm, lidx_hbm, a_hbm, l_hbm,
                        idx_v, rows_v, sem):
    wid = jax.lax.axis_index("s") * NC + jax.lax.axis_index("c")

    @pl.when(wid < NA_W)
    def _agents():
        base = wid * RPW
        pltpu.sync_copy(aidx_hbm.at[pl.ds(base, RPW)], idx_v)
        pltpu.async_copy(y_hbm.at[idx_v], rows_v, sem).wait()
        pltpu.sync_copy(rows_v, a_hbm.at[pl.ds(base, RPW)])

    @pl.when(wid >= NA_W)
    def _lanes():
        base = (wid - NA_W) * RPW
        pltpu.sync_copy(lidx_hbm.at[pl.ds(base, RPW)], idx_v)
        pltpu.async_copy(y_hbm.at[idx_v], rows_v, sem).wait()
        pltpu.sync_copy(rows_v, l_hbm.at[pl.ds(base, RPW)])


_sc_out_gather = functools.partial(
    pl.kernel,
    mesh=_SC_MESH,
    out_type=[
        jax.ShapeDtypeStruct((B * NA_PER, D), jnp.float32),
        jax.ShapeDtypeStruct((B * NL_PER, D), jnp.float32),
    ],
    scratch_types=[
        pltpu.VMEM((RPW,), jnp.int32),
        pltpu.VMEM((RPW, D), jnp.float32),
        pltpu.SemaphoreType.DMA,
    ],
)(_sc_out_gather_body)


def _dot(a, b):
    return jax.lax.dot(a, b, precision=PH)


def _encoder_body(nodes_ref, ids_ref, Wqkv_ref, bqkv_ref,
                  Wo1_ref, bo1_ref, Wo2_ref, W1_ref,
                  gamma_ref, beta_ref, W2_ref,
                  z_out_ref,
                  logits_scr):
    f32 = jnp.float32
    nodes = nodes_ref[:]
    scale = D ** (-0.5)

    iota_r = jax.lax.broadcasted_iota(jnp.int32, (NPS, N), 1)
    iota_c = jax.lax.broadcasted_iota(jnp.int32, (N, NPS), 0)
    P_all = jnp.concatenate(
        [(ids_ref[b, :][:, None] == iota_r).astype(f32) for b in range(B)],
        axis=0)
    Pt_all = jnp.concatenate(
        [(iota_c == ids_ref[b, :][None, :]).astype(f32) for b in range(B)],
        axis=1)

    G = _dot(P_all, nodes)
    QKV = _dot(G, Wqkv_ref[:]) + bqkv_ref[:]
    Qa = QKV[:, 0:H * D]
    Ka = QKV[:, H * D:2 * H * D]
    Va = jax.nn.relu(QKV[:, 2 * H * D:3 * H * D])

    M = jnp.float32(-jnp.inf)
    for b in range(B):
        rows = slice(b * NPS, (b + 1) * NPS)
        for h in range(H):
            sl = slice(h * D, (h + 1) * D)
            lg = jax.lax.dot_general(
                Qa[rows, sl], Ka[rows, sl],
                (((1,), (1,)), ((), ())), precision=PH) * scale
            logits_scr[b, h] = lg
            M = jnp.maximum(M, jnp.max(lg))

    so_rows = []
    for b in range(B):
        rows = slice(b * NPS, (b + 1) * NPS)
        s_cols = []
        o_cols = []
        for h in range(H):
            sl = slice(h * D, (h + 1) * D)
            att = jnp.exp(logits_scr[b, h] - M)
            s_cols.append(jnp.sum(att, axis=1)[:, None])
            o_cols.append(_dot(att, Va[rows, sl]))
        s_cols.append(jnp.zeros((NPS, D - H), f32))
        so_rows.append(jnp.concatenate(s_cols + o_cols, axis=1))
    SO = jnp.concatenate(so_rows, axis=0)

    R = _dot(Pt_all, SO)
    att_sum = R[:, 0:D]
    denom = jnp.maximum(att_sum, jnp.float32(1e-30))
    o_cols = [R[:, D + h * D:D + (h + 1) * D] / denom[:, h:h + 1]
              for h in range(H)]
    O = jnp.concatenate(o_cols, axis=1)

    out = _dot(jax.nn.relu(_dot(O, Wo1_ref[:]) + bo1_ref[:]), Wo2_ref[:])
    x = _dot(nodes, W1_ref[:]) + out
    mu = jnp.mean(x, axis=-1, keepdims=True)
    var = jnp.mean((x - mu) * (x - mu), axis=-1, keepdims=True)
    x = (x - mu) * jax.lax.rsqrt(var + 1e-5) * gamma_ref[:] + beta_ref[:]
    x = jax.nn.relu(x)
    x = _dot(x, W2_ref[:])

    ii = jax.lax.broadcasted_iota(jnp.int32, (D, D), 0)
    jj = jax.lax.broadcasted_iota(jnp.int32, (D, D), 1)
    eye = (ii == jj).astype(f32)
    y = _dot(jnp.concatenate([x, nodes], axis=1),
             jnp.concatenate([eye, eye], axis=0))
    z_out_ref[:] = jnp.maximum(y, 0.0)


@jax.jit
def kernel(agents, agent_ids, lanes, lane_ids, Wq, bq, Wk, bk, Wv, bv,
           Wo1, bo1, Wo2, W1, gamma, beta, W2):
    nodes = jnp.concatenate([agents, lanes], axis=0)
    ids_all = jnp.concatenate(
        [agent_ids, lane_ids + NA], axis=1).astype(jnp.int32)
    Wqkv = jnp.concatenate([Wq, Wk, Wv], axis=1)
    bqkv = jnp.concatenate([bq, bk, bv]).reshape(1, -1)

    z = pl.pallas_call(
        _encoder_body,
        out_shape=jax.ShapeDtypeStruct((N, D), jnp.float32),
        scratch_shapes=[pltpu.VMEM((B, H, NPS, NPS), jnp.float32)],
    )(nodes, ids_all, Wqkv, bqkv,
      Wo1, bo1.reshape(1, -1), Wo2, W1,
      gamma.reshape(1, -1), beta.reshape(1, -1), W2)

    a_idx = ids_all[:, :NA_PER].reshape(-1)
    l_idx = ids_all[:, NA_PER:].reshape(-1)
    a_out, l_out = _sc_out_gather(z, a_idx, l_idx)
    return (a_out, l_out)


# speedup vs baseline: 1.1652x; 1.1652x over previous
import functools

import jax
import jax.numpy as jnp
from jax.experimental import pallas as pl
from jax.experimental.pallas import tpu as pltpu
from jax.experimental.pallas import tpu_sc as plsc

D = 128
H = 6
NA = 384
NL = 640
B = 8
NA_PER = 48
NL_PER = 80
N = NA + NL
NPS = NA_PER + NL_PER
E = B * NPS

PH = jax.lax.Precision.DEFAULT

_SC_INFO = plsc.get_sparse_core_info()
NC = 1
NS = _SC_INFO.num_subcores
NW = NC * NS
RPW = E // NW
NA_W = (B * NA_PER) // RPW
_SC_MESH = plsc.VectorSubcoreMesh(
    core_axis_name="c", subcore_axis_name="s", num_cores=NC)


def _sc_gather_body(table_hbm, idx_hbm, out_hbm, idx_v, rows_v, sem):
    wid = jax.lax.axis_index("s") * NC + jax.lax.axis_index("c")
    base = wid * RPW
    pltpu.sync_copy(idx_hbm.at[pl.ds(base, RPW)], idx_v)
    pltpu.async_copy(table_hbm.at[idx_v], rows_v, sem).wait()
    pltpu.sync_copy(rows_v, out_hbm.at[pl.ds(base, RPW)])


_sc_gather = functools.partial(
    pl.kernel,
    mesh=_SC_MESH,
    out_type=jax.ShapeDtypeStruct((E, D), jnp.float32),
    scratch_types=[
        pltpu.VMEM((RPW,), jnp.int32),
        pltpu.VMEM((RPW, D), jnp.float32),
        pltpu.SemaphoreType.DMA,
    ],
)(_sc_gather_body)


def _dot(a, b):
    return jax.lax.dot(a, b, precision=PH)


def _encoder_body(nodes_ref, g_ref, ids_ref, Wqkv_ref, bqkv_ref,
                  Wo1_ref, bo1_ref, Wo2_ref, W1_ref,
                  gamma_ref, beta_ref, W2_ref,
                  a_out_ref, l_out_ref,
                  logits_scr):
    f32 = jnp.float32
    nodes = nodes_ref[:]
    scale = D ** (-0.5)

    iota_r = jax.lax.broadcasted_iota(jnp.int32, (NPS, N), 1)
    iota_c = jax.lax.broadcasted_iota(jnp.int32, (N, NPS), 0)
    P_all = jnp.concatenate(
        [(ids_ref[b, :][:, None] == iota_r).astype(f32) for b in range(B)],
        axis=0)
    Pt_all = jnp.concatenate(
        [(iota_c == ids_ref[b, :][None, :]).astype(f32) for b in range(B)],
        axis=1)

    QKV = _dot(g_ref[:], Wqkv_ref[:]) + bqkv_ref[:]
    Qa = QKV[:, 0:H * D]
    Ka = QKV[:, H * D:2 * H * D]
    Va = jax.nn.relu(QKV[:, 2 * H * D:3 * H * D])

    M = jnp.float32(-jnp.inf)
    for b in range(B):
        rows = slice(b * NPS, (b + 1) * NPS)
        for h in range(H):
            sl = slice(h * D, (h + 1) * D)
            lg = jax.lax.dot_general(
                Qa[rows, sl], Ka[rows, sl],
                (((1,), (1,)), ((), ())), precision=PH) * scale
            logits_scr[b, h] = lg
            M = jnp.maximum(M, jnp.max(lg))

    so_rows = []
    for b in range(B):
        rows = slice(b * NPS, (b + 1) * NPS)
        s_cols = []
        o_cols = []
        for h in range(H):
            sl = slice(h * D, (h + 1) * D)
            att = jnp.exp(logits_scr[b, h] - M)
            s_cols.append(jnp.sum(att, axis=1)[:, None])
            o_cols.append(_dot(att, Va[rows, sl]))
        s_cols.append(jnp.zeros((NPS, D - H), f32))
        so_rows.append(jnp.concatenate(s_cols + o_cols, axis=1))
    SO = jnp.concatenate(so_rows, axis=0)

    R = _dot(Pt_all, SO)
    att_sum = R[:, 0:D]
    denom = jnp.maximum(att_sum, jnp.float32(1e-30))
    o_cols = [R[:, D + h * D:D + (h + 1) * D] / denom[:, h:h + 1]
              for h in range(H)]
    O = jnp.concatenate(o_cols, axis=1)

    out = _dot(jax.nn.relu(_dot(O, Wo1_ref[:]) + bo1_ref[:]), Wo2_ref[:])
    x = _dot(nodes, W1_ref[:]) + out
    mu = jnp.mean(x, axis=-1, keepdims=True)
    var = jnp.mean((x - mu) * (x - mu), axis=-1, keepdims=True)
    x = (x - mu) * jax.lax.rsqrt(var + 1e-5) * gamma_ref[:] + beta_ref[:]
    x = jax.nn.relu(x)
    x = _dot(x, W2_ref[:])

    ii = jax.lax.broadcasted_iota(jnp.int32, (D, D), 0)
    jj = jax.lax.broadcasted_iota(jnp.int32, (D, D), 1)
    eye = (ii == jj).astype(f32)
    y = _dot(jnp.concatenate([x, nodes], axis=1),
             jnp.concatenate([eye, eye], axis=0))

    ysel = jnp.maximum(_dot(P_all, y), 0.0)
    for b in range(B):
        a_out_ref[b * NA_PER:(b + 1) * NA_PER, :] = \
            ysel[b * NPS:b * NPS + NA_PER, :]
        l_out_ref[b * NL_PER:(b + 1) * NL_PER, :] = \
            ysel[b * NPS + NA_PER:(b + 1) * NPS, :]


@jax.jit
def kernel(agents, agent_ids, lanes, lane_ids, Wq, bq, Wk, bk, Wv, bv,
           Wo1, bo1, Wo2, W1, gamma, beta, W2):
    nodes = jnp.concatenate([agents, lanes], axis=0)
    ids_all = jnp.concatenate(
        [agent_ids, lane_ids + NA], axis=1).astype(jnp.int32)
    Wqkv = jnp.concatenate([Wq, Wk, Wv], axis=1)
    bqkv = jnp.concatenate([bq, bk, bv]).reshape(1, -1)

    G = _sc_gather(nodes, ids_all.reshape(-1))

    out = pl.pallas_call(
        _encoder_body,
        out_shape=[
            jax.ShapeDtypeStruct((B * NA_PER, D), jnp.float32),
            jax.ShapeDtypeStruct((B * NL_PER, D), jnp.float32),
        ],
        scratch_shapes=[pltpu.VMEM((B, H, NPS, NPS), jnp.float32)],
    )(nodes, G, ids_all, Wqkv, bqkv,
      Wo1, bo1.reshape(1, -1), Wo2, W1,
      gamma.reshape(1, -1), beta.reshape(1, -1), W2)
    return (out[0], out[1])


# speedup vs baseline: 1.1683x; 1.0026x over previous
import functools

import jax
import jax.numpy as jnp
from jax.experimental import pallas as pl
from jax.experimental.pallas import tpu as pltpu
from jax.experimental.pallas import tpu_sc as plsc

D = 128
H = 6
NA = 384
NL = 640
B = 8
NA_PER = 48
NL_PER = 80
N = NA + NL
NPS = NA_PER + NL_PER
E = B * NPS

PH = jax.lax.Precision.DEFAULT

_SC_INFO = plsc.get_sparse_core_info()
NC = 1
NS = _SC_INFO.num_subcores
NW = NC * NS
RPW = E // NW
_SC_MESH = plsc.VectorSubcoreMesh(
    core_axis_name="c", subcore_axis_name="s", num_cores=NC)


def _sc_gather_body(table_hbm, idx_hbm, out_hbm, idx_v, rows_v, sem):
    wid = jax.lax.axis_index("s") * NC + jax.lax.axis_index("c")
    base = wid * RPW
    pltpu.sync_copy(idx_hbm.at[pl.ds(base, RPW)], idx_v)
    pltpu.async_copy(table_hbm.at[idx_v], rows_v, sem).wait()
    pltpu.sync_copy(rows_v, out_hbm.at[pl.ds(base, RPW)])


_sc_gather = functools.partial(
    pl.kernel,
    mesh=_SC_MESH,
    out_type=jax.ShapeDtypeStruct((E, D), jnp.float32),
    scratch_types=[
        pltpu.VMEM((RPW,), jnp.int32),
        pltpu.VMEM((RPW, D), jnp.float32),
        pltpu.SemaphoreType.DMA,
    ],
)(_sc_gather_body)


def _dot(a, b):
    return jax.lax.dot(a, b, precision=PH)


def _encoder_body(nodes_ref, g_ref, ids_ref, Wqkv_ref, bqkv_ref,
                  Wo1_ref, bo1_ref, Wo2_ref, W1_ref,
                  gamma_ref, beta_ref, W2_ref,
                  a_out_ref, l_out_ref,
                  logits_scr):
    f32 = jnp.float32
    nodes = nodes_ref[:]
    scale = D ** (-0.5)

    iota_r = jax.lax.broadcasted_iota(jnp.int32, (NPS, N), 1)
    iota_c = jax.lax.broadcasted_iota(jnp.int32, (N, NPS), 0)
    P_all = jnp.concatenate(
        [(ids_ref[b, :][:, None] == iota_r).astype(f32) for b in range(B)],
        axis=0)
    Pt_all = jnp.concatenate(
        [(iota_c == ids_ref[b, :][None, :]).astype(f32) for b in range(B)],
        axis=1)

    QKV = _dot(g_ref[:], Wqkv_ref[:]) + bqkv_ref[:]
    Qa = QKV[:, 0:H * D]
    Ka = QKV[:, H * D:2 * H * D]
    Va = jax.nn.relu(QKV[:, 2 * H * D:3 * H * D])

    M = jnp.float32(-jnp.inf)
    for b in range(B):
        rows = slice(b * NPS, (b + 1) * NPS)
        for h in range(H):
            sl = slice(h * D, (h + 1) * D)
            lg = jax.lax.dot_general(
                Qa[rows, sl], Ka[rows, sl],
                (((1,), (1,)), ((), ())), precision=PH) * scale
            logits_scr[b, h] = lg
            M = jnp.maximum(M, jnp.max(lg))

    so_rows = []
    for b in range(B):
        rows = slice(b * NPS, (b + 1) * NPS)
        s_cols = []
        o_cols = []
        for h in range(H):
            sl = slice(h * D, (h + 1) * D)
            att = jnp.exp(logits_scr[b, h] - M)
            s_cols.append(jnp.sum(att, axis=1)[:, None])
            o_cols.append(_dot(att, Va[rows, sl]))
        s_cols.append(jnp.zeros((NPS, D - H), f32))
        so_rows.append(jnp.concatenate(s_cols + o_cols, axis=1))
    SO = jnp.concatenate(so_rows, axis=0)

    R = _dot(Pt_all, SO)
    att_sum = R[:, 0:D]
    denom = jnp.maximum(att_sum, jnp.float32(1e-30))
    o_cols = [R[:, D + h * D:D + (h + 1) * D] / denom[:, h:h + 1]
              for h in range(H)]
    O = jnp.concatenate(o_cols, axis=1)

    out = _dot(jax.nn.relu(_dot(O, Wo1_ref[:]) + bo1_ref[:]), Wo2_ref[:])
    x = _dot(nodes, W1_ref[:]) + out
    mu = jnp.mean(x, axis=-1, keepdims=True)
    var = jnp.mean((x - mu) * (x - mu), axis=-1, keepdims=True)
    x = (x - mu) * jax.lax.rsqrt(var + 1e-5) * gamma_ref[:] + beta_ref[:]
    x = jax.nn.relu(x)
    x = _dot(x, W2_ref[:])

    ii = jax.lax.broadcasted_iota(jnp.int32, (D, D), 0)
    jj = jax.lax.broadcasted_iota(jnp.int32, (D, D), 1)
    eye = (ii == jj).astype(f32)
    y = _dot(jnp.concatenate([x, nodes], axis=1),
             jnp.concatenate([eye, eye], axis=0))

    ysel = jnp.maximum(_dot(P_all, y), 0.0)
    for b in range(B):
        a_out_ref[b * NA_PER:(b + 1) * NA_PER, :] = \
            ysel[b * NPS:b * NPS + NA_PER, :]
        l_out_ref[b * NL_PER:(b + 1) * NL_PER, :] = \
            ysel[b * NPS + NA_PER:(b + 1) * NPS, :]


@jax.jit
def kernel(agents, agent_ids, lanes, lane_ids, Wq, bq, Wk, bk, Wv, bv,
           Wo1, bo1, Wo2, W1, gamma, beta, W2):
    nodes = jnp.concatenate([agents, lanes], axis=0)
    ids_all = jnp.concatenate(
        [agent_ids, lane_ids + NA], axis=1).astype(jnp.int32)
    Wqkv = jnp.concatenate([Wq, Wk, Wv], axis=1)
    bqkv = jnp.concatenate([bq, bk, bv]).reshape(1, -1)

    G = _sc_gather(nodes, ids_all.reshape(-1))

    out = pl.pallas_call(
        _encoder_body,
        out_shape=[
            jax.ShapeDtypeStruct((B * NA_PER, D), jnp.float32),
            jax.ShapeDtypeStruct((B * NL_PER, D), jnp.float32),
        ],
        scratch_shapes=[pltpu.VMEM((B, H, NPS, NPS), jnp.float32)],
    )(nodes, G, ids_all, Wqkv, bqkv,
      Wo1, bo1.reshape(1, -1), Wo2, W1,
      gamma.reshape(1, -1), beta.reshape(1, -1), W2)
    return (out[0], out[1])
